# baseline (speedup 1.0000x reference)
import numpy as np
import jax
import jax.numpy as jnp
from jax.experimental import pallas as pl
from jax.experimental.pallas import tpu as pltpu

_B = 8
_NG = 8
_NN = 256
_SIZE = 32
_D = 32
_NL = 5
_TI = 64
_JG = _NN // 4


def _goal_pos():
    gi = np.linspace(0, _SIZE * _SIZE - 1, _NG).astype(np.int64)
    gx = (gi % _SIZE).astype(np.float32)
    gy = (gi // _SIZE).astype(np.float32)
    return np.stack([gx / _SIZE * 2 - 1, gy / _SIZE * 2 - 1], -1).astype(np.float32)


_GOAL_POS = _goal_pos()
_SP32 = np.kron(np.ones((1, 4)), np.eye(_D)).astype(np.float32)
_PLACE = np.stack([np.kron(np.eye(4)[k:k + 1], np.eye(_D))
                   for k in range(4)]).astype(np.float32)
_F_FOLD = np.kron(np.ones((4, 1)), np.eye(_D)).astype(np.float32)
_SEL = np.zeros((4, _JG, _NN), dtype=np.float32)
for _k in range(4):
    for _jg in range(_JG):
        _SEL[_k, _jg, 4 * _jg + _k] = 1.0
_SELT = np.transpose(_SEL, (0, 2, 1)).copy()
_PICK = np.zeros((4, 128, 1), dtype=np.float32)
for _k in range(4):
    _PICK[_k, 32 * _k, 0] = 1.0
_SUMSEL = np.kron(np.eye(_TI), np.ones((1, _JG))).astype(np.float32)


def _sigmoid(x):
    return 0.5 * jnp.tanh(0.5 * x) + 0.5


def _silu(x):
    y = 0.5 * x
    return y * jnp.tanh(y) + y


def _fwd_kernel(p0_ref, te_ref, h0_ref, sp_ref, pl_ref, f_ref, sel_ref,
                selt_ref, pick_ref, sum_ref,
                wxi_ref, wxj_ref, wt_ref, wpl_ref, m1b_ref,
                m2_ref, m2b_ref, aw_ref, ab_ref, p1_ref, p1b_ref,
                p2w_ref, p2b_ref, n1h_ref, n1a_ref, n1b_ref, n2_ref, n2b_ref,
                tw_ref, tb_ref, out_ref):
    h = h0_ref[...]
    p = p0_ref[0]
    te = te_ref[0]
    fm = f_ref[...]
    px = p[:, 0:1]
    py = p[:, 1:2]
    pxr = px.reshape(1, _NN)
    pyr = py.reshape(1, _NN)

    for l in range(_NL):
        ai_all = h @ wxi_ref[l] + (te @ wt_ref[l]) + m1b_ref[l]
        ai4 = ai_all @ sp_ref[...]
        aj_all = h @ wxj_ref[l]
        aj_pack = ((sel_ref[0] @ aj_all) @ pl_ref[0]
                   + (sel_ref[1] @ aj_all) @ pl_ref[1]
                   + (sel_ref[2] @ aj_all) @ pl_ref[2]
                   + (sel_ref[3] @ aj_all) @ pl_ref[3])
        wpl = wpl_ref[l]
        m2w = m2_ref[l]
        m2b = m2b_ref[l]
        aw = aw_ref[l]
        ab = ab_ref[l][0]
        p1w = p1_ref[l]
        p1b = p1b_ref[l]
        p2w = p2w_ref[l]
        p2b = p2b_ref[l][0]

        aggr_h_parts = []
        apx_parts = []
        apy_parts = []
        for c in range(_NN // _TI):
            i0 = c * _TI
            px_c = px[i0:i0 + _TI]
            py_c = py[i0:i0 + _TI]
            dx = px_c - pxr
            dy = py_c - pyr
            radial = dx * dx + dy * dy
            norm = jnp.sqrt(radial) + 1e-6
            dxn = dx / norm
            dyn = dy / norm
            rp = ((radial @ selt_ref[0])[:, :, None] * wpl[0:1].reshape(1, 1, 128)
                  + (radial @ selt_ref[1])[:, :, None] * wpl[1:2].reshape(1, 1, 128)
                  + (radial @ selt_ref[2])[:, :, None] * wpl[2:3].reshape(1, 1, 128)
                  + (radial @ selt_ref[3])[:, :, None] * wpl[3:4].reshape(1, 1, 128))
            pre = (ai4[i0:i0 + _TI][:, None, :] + aj_pack[None, :, :]
                   + rp)
            m = _silu(pre).reshape(_TI * _JG, 128)
            mb = m.astype(jnp.bfloat16)
            m = _silu(jnp.dot(mb, m2w, preferred_element_type=jnp.float32)
                      + m2b)
            mb = m.astype(jnp.bfloat16)
            m = m * _sigmoid(jnp.dot(mb, aw, preferred_element_type=jnp.float32)
                             + ab)
            t1 = _silu(jnp.dot(m.astype(jnp.bfloat16), p1w,
                               preferred_element_type=jnp.float32) + p1b)
            pv128 = jnp.dot(t1.astype(jnp.bfloat16), p2w,
                            preferred_element_type=jnp.float32) + p2b
            pv2 = ((pv128 @ pick_ref[0]).reshape(_TI, _JG) @ sel_ref[0]
                   + (pv128 @ pick_ref[1]).reshape(_TI, _JG) @ sel_ref[1]
                   + (pv128 @ pick_ref[2]).reshape(_TI, _JG) @ sel_ref[2]
                   + (pv128 @ pick_ref[3]).reshape(_TI, _JG) @ sel_ref[3])
            aggr_h_parts.append((sum_ref[...] @ m) @ fm)
            apx_parts.append(jnp.sum(dxn * pv2, axis=1, keepdims=True))
            apy_parts.append(jnp.sum(dyn * pv2, axis=1, keepdims=True))
        aggr_h = jnp.concatenate(aggr_h_parts, axis=0)
        apx = jnp.concatenate(apx_parts, axis=0)
        apy = jnp.concatenate(apy_parts, axis=0)

        if l < _NL - 1:
            u = h @ n1h_ref[l] + aggr_h @ n1a_ref[l] + n1b_ref[l]
            u = _silu(u)
            u = u @ n2_ref[l] + n2b_ref[l]
            u = jnp.where(u >= 0, u, 0.01 * u)
            h = h + u
            te = _silu(te @ tw_ref[l] + tb_ref[l])

        px = px + apx * (1.0 / _NN)
        py = py + apy * (1.0 / _NN)
        pxr = px.reshape(1, _NN)
        pyr = py.reshape(1, _NN)

    out_ref[0] = jnp.concatenate([px[_NG:], py[_NG:]], axis=1)


def kernel(pos, timesteps, params):
    goal = jnp.asarray(_GOAL_POS)
    p0 = jnp.concatenate(
        [jnp.broadcast_to(goal[None], (_B, _NG, 2)), pos], axis=1)

    t = timesteps.astype(jnp.float32)
    half = _D // 2
    freqs = jnp.exp(-np.log(10000.0) * jnp.arange(half, dtype=jnp.float32) / half)
    args = t[:, None] * freqs[None, :]
    temb = jnp.concatenate([jnp.cos(args), jnp.sin(args)], axis=-1)
    temb = temb.reshape(_B, 1, _D)

    hw = params["h_in"]["w"]
    hb = params["h_in"]["b"]
    hg = hw[:, 0] + hb
    hs = hw[:, 1] + hb
    h0 = jnp.where(jnp.arange(_NN)[:, None] < _NG, hg[None, :], hs[None, :])

    lyr = params["layers"]
    eye4 = jnp.eye(4, dtype=jnp.float32)
    ones32 = jnp.ones((1, _D), dtype=jnp.float32)

    def stk(f):
        return jnp.stack([f(lp) for lp in lyr])

    def stk4(f):
        return jnp.stack([f(lp) for lp in lyr[:_NL - 1]])

    wxiT = stk(lambda lp: lp["m1"]["w"][:, 0:_D].T)
    wxjT = stk(lambda lp: lp["m1"]["w"][:, _D:2 * _D].T)
    wpl = stk(lambda lp: jnp.kron(eye4,
                                  lp["m1"]["w"][:, 2 * _D:2 * _D + 1].T))
    wtT = stk(lambda lp: lp["m1"]["w"][:, 2 * _D + 1:].T)
    m1b = stk(lambda lp: lp["m1"]["b"][None, :])
    m2B = stk(lambda lp: jnp.kron(eye4, lp["m2"]["w"].T)
              .astype(jnp.bfloat16))
    m2b = stk(lambda lp: jnp.tile(lp["m2"]["b"][None, :], (1, 4)))
    aw4 = stk(lambda lp: jnp.kron(eye4, lp["a"]["w"].T @ ones32)
              .astype(jnp.bfloat16))
    ab = stk(lambda lp: lp["a"]["b"][None, :])
    p1B = stk(lambda lp: jnp.kron(eye4, lp["p1"]["w"].T)
              .astype(jnp.bfloat16))
    p1b = stk(lambda lp: jnp.tile(lp["p1"]["b"][None, :], (1, 4)))
    p2w4 = stk(lambda lp: jnp.kron(eye4, lp["p2"]["w"].T @ ones32)
               .astype(jnp.bfloat16))
    p2b = stk(lambda lp: lp["p2"]["b"][None, :])
    n1hT = stk4(lambda lp: lp["n1"]["w"][:, 0:_D].T)
    n1aT = stk4(lambda lp: lp["n1"]["w"][:, _D:].T)
    n1b = stk4(lambda lp: lp["n1"]["b"][None, :])
    n2T = stk4(lambda lp: lp["n2"]["w"].T)
    n2b = stk4(lambda lp: lp["n2"]["b"][None, :])
    tT = jnp.stack([tl["w"].T for tl in params["t_layers"]])
    tb = jnp.stack([tl["b"][None, :] for tl in params["t_layers"]])

    def full(shape):
        return pl.BlockSpec(shape, lambda b: (0,) * len(shape))

    in_specs = [
        pl.BlockSpec((1, _NN, 2), lambda b: (b, 0, 0)),
        pl.BlockSpec((1, 1, _D), lambda b: (b, 0, 0)),
        full((_NN, _D)),
        full((_D, 128)), full((4, _D, 128)), full((128, _D)),
        full((4, _JG, _NN)), full((4, _NN, _JG)), full((4, 128, 1)),
        full((_TI, _TI * _JG)),
        full((_NL, _D, _D)), full((_NL, _D, _D)), full((_NL, _D, _D)),
        full((_NL, 4, 128)), full((_NL, 1, _D)),
        full((_NL, 128, 128)), full((_NL, 1, 128)),
        full((_NL, 128, 128)), full((_NL, 1, 1)),
        full((_NL, 128, 128)), full((_NL, 1, 128)),
        full((_NL, 128, 128)), full((_NL, 1, 1)),
        full((_NL - 1, _D, _D)), full((_NL - 1, _D, _D)), full((_NL - 1, 1, _D)),
        full((_NL - 1, _D, _D)), full((_NL - 1, 1, _D)),
        full((_NL - 1, _D, _D)), full((_NL - 1, 1, _D)),
    ]

    out = pl.pallas_call(
        _fwd_kernel,
        grid=(_B,),
        in_specs=in_specs,
        out_specs=pl.BlockSpec((1, _NN - _NG, 2), lambda b: (b, 0, 0)),
        out_shape=jax.ShapeDtypeStruct((_B, _NN - _NG, 2), jnp.float32),
        compiler_params=pltpu.CompilerParams(dimension_semantics=("parallel",)),
    )(p0, temb, h0, jnp.asarray(_SP32), jnp.asarray(_PLACE),
      jnp.asarray(_F_FOLD), jnp.asarray(_SEL), jnp.asarray(_SELT),
      jnp.asarray(_PICK), jnp.asarray(_SUMSEL),
      wxiT, wxjT, wtT, wpl, m1b, m2B, m2b, aw4, ab,
      p1B, p1b, p2w4, p2b, n1hT, n1aT, n1b, n2T, n2b, tT, tb)
    return out


# speedup vs baseline: 1.0973x; 1.0973x over previous
import numpy as np
import jax
import jax.numpy as jnp
from jax.experimental import pallas as pl
from jax.experimental.pallas import tpu as pltpu

_B = 8
_NG = 8
_NN = 256
_SIZE = 32
_D = 32
_NL = 5
_TI = 128
_JG = _NN // 4


def _goal_pos():
    gi = np.linspace(0, _SIZE * _SIZE - 1, _NG).astype(np.int64)
    gx = (gi % _SIZE).astype(np.float32)
    gy = (gi // _SIZE).astype(np.float32)
    return np.stack([gx / _SIZE * 2 - 1, gy / _SIZE * 2 - 1], -1).astype(np.float32)


_GOAL_POS = _goal_pos()
_SP32 = np.kron(np.ones((1, 4)), np.eye(_D)).astype(np.float32)
_PLACE = np.stack([np.kron(np.eye(4)[k:k + 1], np.eye(_D))
                   for k in range(4)]).astype(np.float32)
_F_FOLD = np.kron(np.ones((4, 1)), np.eye(_D)).astype(np.float32)
_SEL = np.zeros((4, _JG, _NN), dtype=np.float32)
for _k in range(4):
    for _jg in range(_JG):
        _SEL[_k, _jg, 4 * _jg + _k] = 1.0
_SELT = np.transpose(_SEL, (0, 2, 1)).copy()
_PICK = np.zeros((4, 128, 1), dtype=np.float32)
for _k in range(4):
    _PICK[_k, 32 * _k, 0] = 1.0
_SUMSEL = np.kron(np.eye(_TI), np.ones((1, _JG))).astype(np.float32)


def _sigmoid(x):
    return 0.5 * jnp.tanh(0.5 * x) + 0.5


def _silu(x):
    y = 0.5 * x
    return y * jnp.tanh(y) + y


def _silu_pre(y):
    t = jnp.tanh(y)
    return y * t + y


def _fwd_kernel(p0_ref, te_ref, h0_ref, sp_ref, pl_ref, f_ref, sel_ref,
                selt_ref, pick_ref, sum_ref,
                wxi_ref, wxj_ref, wt_ref, wpl_ref, m1b_ref,
                m2_ref, m2b_ref, aw_ref, ab_ref, p1_ref, p1b_ref,
                p2w_ref, p2b_ref, n1h_ref, n1a_ref, n1b_ref, n2_ref, n2b_ref,
                tw_ref, tb_ref, out_ref):
    h = h0_ref[...]
    p = p0_ref[0]
    te = te_ref[0]
    fm = f_ref[...]
    px = p[:, 0:1]
    py = p[:, 1:2]
    pxr = px.reshape(1, _NN)
    pyr = py.reshape(1, _NN)

    for l in range(_NL):
        ai_all = h @ wxi_ref[l] + (te @ wt_ref[l]) + m1b_ref[l]
        ai4 = ai_all @ sp_ref[...]
        aj_all = h @ wxj_ref[l]
        aj_pack = ((sel_ref[0] @ aj_all) @ pl_ref[0]
                   + (sel_ref[1] @ aj_all) @ pl_ref[1]
                   + (sel_ref[2] @ aj_all) @ pl_ref[2]
                   + (sel_ref[3] @ aj_all) @ pl_ref[3])
        wpl = wpl_ref[l]
        m2w = m2_ref[l]
        m2b = m2b_ref[l]
        aw = aw_ref[l]
        ab = ab_ref[l][0]
        p1w = p1_ref[l]
        p1b = p1b_ref[l]
        p2w = p2w_ref[l]
        p2b = p2b_ref[l][0]

        aggr_h_parts = []
        apx_parts = []
        apy_parts = []
        for c in range(_NN // _TI):
            i0 = c * _TI
            px_c = px[i0:i0 + _TI]
            py_c = py[i0:i0 + _TI]
            dx = px_c - pxr
            dy = py_c - pyr
            radial = dx * dx + dy * dy
            norm = jnp.sqrt(radial) + 1e-6
            dxn = dx / norm
            dyn = dy / norm
            rp = ((radial @ selt_ref[0])[:, :, None] * wpl[0:1].reshape(1, 1, 128)
                  + (radial @ selt_ref[1])[:, :, None] * wpl[1:2].reshape(1, 1, 128)
                  + (radial @ selt_ref[2])[:, :, None] * wpl[2:3].reshape(1, 1, 128)
                  + (radial @ selt_ref[3])[:, :, None] * wpl[3:4].reshape(1, 1, 128))
            pre = (ai4[i0:i0 + _TI][:, None, :] + aj_pack[None, :, :]
                   + rp)
            m = _silu_pre(pre).reshape(_TI * _JG, 128)
            m = _silu_pre(m @ m2w + m2b)
            t = jnp.tanh(m @ aw + ab)
            m2x = m * t + m
            t1 = _silu_pre(m2x @ p1w + p1b)
            pv128 = t1 @ p2w + p2b
            pv2 = ((pv128 @ pick_ref[0]).reshape(_TI, _JG) @ sel_ref[0]
                   + (pv128 @ pick_ref[1]).reshape(_TI, _JG) @ sel_ref[1]
                   + (pv128 @ pick_ref[2]).reshape(_TI, _JG) @ sel_ref[2]
                   + (pv128 @ pick_ref[3]).reshape(_TI, _JG) @ sel_ref[3])
            aggr_h_parts.append((sum_ref[...] @ m2x) @ fm)
            apx_parts.append(jnp.sum(dxn * pv2, axis=1, keepdims=True))
            apy_parts.append(jnp.sum(dyn * pv2, axis=1, keepdims=True))
        aggr_h = jnp.concatenate(aggr_h_parts, axis=0)
        apx = jnp.concatenate(apx_parts, axis=0)
        apy = jnp.concatenate(apy_parts, axis=0)

        if l < _NL - 1:
            u = h @ n1h_ref[l] + aggr_h @ n1a_ref[l] + n1b_ref[l]
            u = _silu(u)
            u = u @ n2_ref[l] + n2b_ref[l]
            u = jnp.where(u >= 0, u, 0.01 * u)
            h = h + u
            te = _silu(te @ tw_ref[l] + tb_ref[l])

        px = px + apx * (1.0 / _NN)
        py = py + apy * (1.0 / _NN)
        pxr = px.reshape(1, _NN)
        pyr = py.reshape(1, _NN)

    out_ref[0] = jnp.concatenate([px[_NG:], py[_NG:]], axis=1)


def kernel(pos, timesteps, params):
    goal = jnp.asarray(_GOAL_POS)
    p0 = jnp.concatenate(
        [jnp.broadcast_to(goal[None], (_B, _NG, 2)), pos], axis=1)

    t = timesteps.astype(jnp.float32)
    half = _D // 2
    freqs = jnp.exp(-np.log(10000.0) * jnp.arange(half, dtype=jnp.float32) / half)
    args = t[:, None] * freqs[None, :]
    temb = jnp.concatenate([jnp.cos(args), jnp.sin(args)], axis=-1)
    temb = temb.reshape(_B, 1, _D)

    hw = params["h_in"]["w"]
    hb = params["h_in"]["b"]
    hg = hw[:, 0] + hb
    hs = hw[:, 1] + hb
    h0 = jnp.where(jnp.arange(_NN)[:, None] < _NG, hg[None, :], hs[None, :])

    lyr = params["layers"]
    eye4 = jnp.eye(4, dtype=jnp.float32)
    ones32 = jnp.ones((1, _D), dtype=jnp.float32)

    def stk(f):
        return jnp.stack([f(lp) for lp in lyr])

    def stk4(f):
        return jnp.stack([f(lp) for lp in lyr[:_NL - 1]])

    wxiT = stk(lambda lp: 0.5 * lp["m1"]["w"][:, 0:_D].T)
    wxjT = stk(lambda lp: 0.5 * lp["m1"]["w"][:, _D:2 * _D].T)
    wpl = stk(lambda lp: 0.5 * jnp.kron(eye4,
                                        lp["m1"]["w"][:, 2 * _D:2 * _D + 1].T))
    wtT = stk(lambda lp: 0.5 * lp["m1"]["w"][:, 2 * _D + 1:].T)
    m1b = stk(lambda lp: 0.5 * lp["m1"]["b"][None, :])
    m2B = stk(lambda lp: 0.5 * jnp.kron(eye4, lp["m2"]["w"].T))
    m2b = stk(lambda lp: 0.5 * jnp.tile(lp["m2"]["b"][None, :], (1, 4)))
    aw4 = stk(lambda lp: 0.5 * jnp.kron(eye4, lp["a"]["w"].T @ ones32))
    ab = stk(lambda lp: 0.5 * lp["a"]["b"][None, :])
    p1B = stk(lambda lp: 0.25 * jnp.kron(eye4, lp["p1"]["w"].T))
    p1b = stk(lambda lp: 0.5 * jnp.tile(lp["p1"]["b"][None, :], (1, 4)))
    p2w4 = stk(lambda lp: jnp.kron(eye4, lp["p2"]["w"].T @ ones32))
    p2b = stk(lambda lp: lp["p2"]["b"][None, :])
    n1hT = stk4(lambda lp: lp["n1"]["w"][:, 0:_D].T)
    n1aT = stk4(lambda lp: lp["n1"]["w"][:, _D:].T)
    n1b = stk4(lambda lp: lp["n1"]["b"][None, :])
    n2T = stk4(lambda lp: lp["n2"]["w"].T)
    n2b = stk4(lambda lp: lp["n2"]["b"][None, :])
    tT = jnp.stack([tl["w"].T for tl in params["t_layers"]])
    tb = jnp.stack([tl["b"][None, :] for tl in params["t_layers"]])

    def full(shape):
        return pl.BlockSpec(shape, lambda b: (0,) * len(shape))

    in_specs = [
        pl.BlockSpec((1, _NN, 2), lambda b: (b, 0, 0)),
        pl.BlockSpec((1, 1, _D), lambda b: (b, 0, 0)),
        full((_NN, _D)),
        full((_D, 128)), full((4, _D, 128)), full((128, _D)),
        full((4, _JG, _NN)), full((4, _NN, _JG)), full((4, 128, 1)),
        full((_TI, _TI * _JG)),
        full((_NL, _D, _D)), full((_NL, _D, _D)), full((_NL, _D, _D)),
        full((_NL, 4, 128)), full((_NL, 1, _D)),
        full((_NL, 128, 128)), full((_NL, 1, 128)),
        full((_NL, 128, 128)), full((_NL, 1, 1)),
        full((_NL, 128, 128)), full((_NL, 1, 128)),
        full((_NL, 128, 128)), full((_NL, 1, 1)),
        full((_NL - 1, _D, _D)), full((_NL - 1, _D, _D)), full((_NL - 1, 1, _D)),
        full((_NL - 1, _D, _D)), full((_NL - 1, 1, _D)),
        full((_NL - 1, _D, _D)), full((_NL - 1, 1, _D)),
    ]

    out = pl.pallas_call(
        _fwd_kernel,
        grid=(_B,),
        in_specs=in_specs,
        out_specs=pl.BlockSpec((1, _NN - _NG, 2), lambda b: (b, 0, 0)),
        out_shape=jax.ShapeDtypeStruct((_B, _NN - _NG, 2), jnp.float32),
        compiler_params=pltpu.CompilerParams(dimension_semantics=("parallel",)),
    )(p0, temb, h0, jnp.asarray(_SP32), jnp.asarray(_PLACE),
      jnp.asarray(0.5 * _F_FOLD), jnp.asarray(_SEL), jnp.asarray(_SELT),
      jnp.asarray(_PICK), jnp.asarray(_SUMSEL),
      wxiT, wxjT, wtT, wpl, m1b, m2B, m2b, aw4, ab,
      p1B, p1b, p2w4, p2b, n1hT, n1aT, n1b, n2T, n2b, tT, tb)
    return out
